# baseline (speedup 1.0000x reference)
import functools

import jax
import jax.numpy as jnp
from jax import lax
from jax.experimental import pallas as pl
from jax.experimental.pallas import tpu as pltpu
from jax.experimental.pallas import tpu_sc as plsc

_EMB = 1024
_F4 = 32
_W = 48
_N = 16384
_NCOORD = 4
_SLICES = 8
_PW = _N // _SLICES
_L = 16
_G = _PW // _L
_NCHUNK = 4
_CROWS = _PW // _NCHUNK

_mesh = plsc.VectorSubcoreMesh(core_axis_name="c", subcore_axis_name="s")


@functools.partial(
    pl.kernel,
    out_type=jax.ShapeDtypeStruct((_N, _NCOORD * _F4), jnp.float32),
    mesh=_mesh,
    scratch_types=[
        pltpu.VMEM((_PW,), jnp.float32),
        pltpu.VMEM((_EMB, _W), jnp.float32),
        pltpu.VMEM((_CROWS, _W), jnp.float32),
        pltpu.VMEM((_CROWS, _W), jnp.float32),
        pltpu.SemaphoreType.DMA,
        pltpu.SemaphoreType.DMA,
        pltpu.SemaphoreType.DMA,
        pltpu.SemaphoreType.DMA,
    ],
    compiler_params=pltpu.CompilerParams(
        use_tc_tiling_on_sc=False, needs_layout_passes=False
    ),
)
def _emb_kernel(pos_hbm, table_hbm, out_hbm, pos_v, tq_v, out0_v, out1_v,
                sem0, sem1, sem2, sem3):
    wid = lax.axis_index("s") * 2 + lax.axis_index("c")
    j = wid % _NCOORD
    sl = wid // _NCOORD
    base = sl * _PW

    tdma = pltpu.async_copy(
        table_hbm.at[:, pl.ds(j * _F4, _F4)], tq_v.at[:, pl.ds(0, _F4)], sem0
    )
    t2dma = pltpu.async_copy(
        table_hbm.at[:, pl.ds(j * _F4, _W - _F4)],
        tq_v.at[:, pl.ds(_F4, _W - _F4)],
        sem2,
    )
    pdma = pltpu.async_copy(pos_hbm.at[j, pl.ds(base, _PW)], pos_v, sem1)
    tdma.wait()
    t2dma.wait()
    pdma.wait()

    lanes = lax.iota(jnp.int32, _L)

    handles = []
    for c in range(_NCHUNK):
        buf, sem = ((out0_v, sem2), (out1_v, sem3))[c % 2]
        if c >= 2:
            handles[c - 2].wait()

        @plsc.parallel_loop(0, _G // _NCHUNK, step=1, unroll=4)
        def body(g, _c=c, _buf=buf):
            row = g * _L + lanes
            pv = pos_v[pl.ds(_c * _CROWS + g * _L, _L)]
            d = pv * float(_EMB)
            dc = jnp.minimum(jnp.maximum(d, 0.0), float(_EMB - 1))
            li = dc.astype(jnp.int32)
            lw = d - li.astype(jnp.float32)
            rw = 1.0 - lw
            ri = jnp.minimum(li + 1, _EMB - 1)
            for k in range(_F4):
                col = lanes + k
                lv = plsc.load_gather(tq_v, [li, col])
                rv = plsc.load_gather(tq_v, [ri, col])
                o = lw * lv + rw * rv
                plsc.store_scatter(_buf, [row, col & (_F4 - 1)], o)

        handles.append(
            pltpu.async_copy(
                buf.at[:, pl.ds(0, _F4)],
                out_hbm.at[pl.ds(base + c * _CROWS, _CROWS), pl.ds(j * _F4, _F4)],
                sem,
            )
        )
    handles[-2].wait()
    handles[-1].wait()


def kernel(seq_positions, lookup_weight):
    pos_t = seq_positions.T
    tab128 = lookup_weight.reshape(_EMB, _NCOORD * _F4)
    return _emb_kernel(pos_t, tab128)


# speedup vs baseline: 1.4556x; 1.4556x over previous
import functools

import jax
import jax.numpy as jnp
from jax import lax
from jax.experimental import pallas as pl
from jax.experimental.pallas import tpu as pltpu
from jax.experimental.pallas import tpu_sc as plsc

_EMB = 1024
_F4 = 32
_N = 16384
_NCOORD = 4
_SLICES = 8
_PW = _N // _SLICES
_L = 16
_G = _PW // _L

_mesh = plsc.VectorSubcoreMesh(core_axis_name="c", subcore_axis_name="s")


@functools.partial(
    pl.kernel,
    out_type=jax.ShapeDtypeStruct((_N, _NCOORD * _F4), jnp.float32),
    mesh=_mesh,
    scratch_types=[
        pltpu.VMEM((_PW,), jnp.float32),
        pltpu.VMEM((_EMB, _F4), jnp.float32),
        pltpu.VMEM((_PW // 4, _F4), jnp.float32),
        pltpu.VMEM((_PW // 4, _F4), jnp.float32),
        pltpu.SemaphoreType.DMA,
        pltpu.SemaphoreType.DMA,
        pltpu.SemaphoreType.DMA,
        pltpu.SemaphoreType.DMA,
    ],
    compiler_params=pltpu.CompilerParams(
        use_tc_tiling_on_sc=False, needs_layout_passes=False
    ),
)
def _emb_kernel(pos_hbm, table_hbm, out_hbm, pos_v, tq_v, out0_v, out1_v, sem0, sem1, sem2, sem3):
    wid = lax.axis_index("s") * 2 + lax.axis_index("c")
    j = wid % _NCOORD
    sl = wid // _NCOORD
    base = sl * _PW

    tdma = pltpu.async_copy(table_hbm.at[:, pl.ds(j * _F4, _F4)], tq_v, sem0)
    pdma = pltpu.async_copy(pos_hbm.at[j, pl.ds(base, _PW)], pos_v, sem1)
    tdma.wait()
    pdma.wait()

    lanes = lax.iota(jnp.int32, _L)
    half = _PW // 4

    handles = []
    for c in range(4):
        buf, sem = ((out0_v, sem2), (out1_v, sem3))[c % 2]
        if c >= 2:
            handles[c - 2].wait()

        @plsc.parallel_loop(0, _G // 4, step=1, unroll=4)
        def body(g, _c=c, _buf=buf):
            row = g * _L + lanes
            pv = pos_v[pl.ds(_c * half + g * _L, _L)]
            d = pv * float(_EMB)
            dc = jnp.minimum(jnp.maximum(d, 0.0), float(_EMB - 1))
            li = dc.astype(jnp.int32)
            lw = d - li.astype(jnp.float32)
            rw = 1.0 - lw
            ri = jnp.minimum(li + 1, _EMB - 1)
            for k in range(_F4):
                kv = lanes ^ k
                lv = plsc.load_gather(tq_v, [li, kv])
                rv = plsc.load_gather(tq_v, [ri, kv])
                o = lw * lv + rw * rv
                plsc.store_scatter(_buf, [row, kv], o)

        handles.append(
            pltpu.async_copy(
                buf,
                out_hbm.at[pl.ds(base + c * half, half), pl.ds(j * _F4, _F4)],
                sem,
            )
        )
    handles[-2].wait()
    handles[-1].wait()


def kernel(seq_positions, lookup_weight):
    pos_t = seq_positions.T
    tab128 = lookup_weight.reshape(_EMB, _NCOORD * _F4)
    return _emb_kernel(pos_t, tab128)


# speedup vs baseline: 1.5255x; 1.0480x over previous
import functools

import jax
import jax.numpy as jnp
from jax import lax
from jax.experimental import pallas as pl
from jax.experimental.pallas import tpu as pltpu
from jax.experimental.pallas import tpu_sc as plsc

_EMB = 1024
_F4 = 32
_N = 16384
_NCOORD = 4
_SLICES = 8
_PW = _N // _SLICES
_L = 16
_G = _PW // _L

_mesh = plsc.VectorSubcoreMesh(core_axis_name="c", subcore_axis_name="s")


@functools.partial(
    pl.kernel,
    out_type=jax.ShapeDtypeStruct((_N, _NCOORD * _F4), jnp.float32),
    mesh=_mesh,
    scratch_types=[
        pltpu.VMEM((_PW,), jnp.float32),
        pltpu.VMEM((_EMB, _F4), jnp.float32),
        pltpu.VMEM((_PW // 2, _F4), jnp.float32),
        pltpu.VMEM((_PW // 2, _F4), jnp.float32),
        pltpu.SemaphoreType.DMA,
        pltpu.SemaphoreType.DMA,
        pltpu.SemaphoreType.DMA,
        pltpu.SemaphoreType.DMA,
    ],
    compiler_params=pltpu.CompilerParams(
        use_tc_tiling_on_sc=False, needs_layout_passes=False
    ),
)
def _emb_kernel(pos_hbm, table_hbm, out_hbm, pos_v, tq_v, out0_v, out1_v, sem0, sem1, sem2, sem3):
    wid = lax.axis_index("s") * 2 + lax.axis_index("c")
    j = wid % _NCOORD
    sl = wid // _NCOORD
    base = sl * _PW

    tdma = pltpu.async_copy(table_hbm.at[:, pl.ds(j * _F4, _F4)], tq_v, sem0)
    pdma = pltpu.async_copy(pos_hbm.at[j, pl.ds(base, _PW)], pos_v, sem1)
    tdma.wait()
    pdma.wait()

    lanes = lax.iota(jnp.int32, _L)
    half = _PW // 2

    handles = []
    for c, (buf, sem) in enumerate(((out0_v, sem2), (out1_v, sem3))):

        @plsc.parallel_loop(0, _G // 2, step=1, unroll=4)
        def body(g, _c=c, _buf=buf):
            row = g * _L + lanes
            pv = pos_v[pl.ds(_c * half + g * _L, _L)]
            d = pv * float(_EMB)
            dc = jnp.minimum(jnp.maximum(d, 0.0), float(_EMB - 1))
            li = dc.astype(jnp.int32)
            lw = d - li.astype(jnp.float32)
            rw = 1.0 - lw
            ri = jnp.minimum(li + 1, _EMB - 1)
            for k in range(0, _F4, 2):
                kv0 = lanes ^ k
                kv1 = lanes ^ (k + 1)
                lv0 = plsc.load_gather(tq_v, [li, kv0])
                rv0 = plsc.load_gather(tq_v, [ri, kv0])
                lv1 = plsc.load_gather(tq_v, [li, kv1])
                rv1 = plsc.load_gather(tq_v, [ri, kv1])
                o0 = lw * lv0 + rw * rv0
                o1 = lw * lv1 + rw * rv1
                plsc.store_scatter(_buf, [row, kv0], o0)
                plsc.store_scatter(_buf, [row, kv1], o1)

        handles.append(
            pltpu.async_copy(
                buf,
                out_hbm.at[pl.ds(base + c * half, half), pl.ds(j * _F4, _F4)],
                sem,
            )
        )
    for h in handles:
        h.wait()


def kernel(seq_positions, lookup_weight):
    pos_t = seq_positions.T
    tab128 = lookup_weight.reshape(_EMB, _NCOORD * _F4)
    return _emb_kernel(pos_t, tab128)


# speedup vs baseline: 1.5402x; 1.0096x over previous
import functools

import jax
import jax.numpy as jnp
from jax import lax
from jax.experimental import pallas as pl
from jax.experimental.pallas import tpu as pltpu
from jax.experimental.pallas import tpu_sc as plsc

_EMB = 1024
_F4 = 32
_N = 16384
_NCOORD = 4
_SLICES = 8
_PW = _N // _SLICES
_L = 16
_G = _PW // _L

_mesh = plsc.VectorSubcoreMesh(core_axis_name="c", subcore_axis_name="s")


@functools.partial(
    pl.kernel,
    out_type=jax.ShapeDtypeStruct((_N, _NCOORD * _F4), jnp.float32),
    mesh=_mesh,
    scratch_types=[
        pltpu.VMEM((_PW,), jnp.float32),
        pltpu.VMEM((_EMB, _F4), jnp.float32),
        pltpu.VMEM((_PW // 2, _F4), jnp.float32),
        pltpu.VMEM((_PW // 2, _F4), jnp.float32),
        pltpu.SemaphoreType.DMA,
        pltpu.SemaphoreType.DMA,
        pltpu.SemaphoreType.DMA,
        pltpu.SemaphoreType.DMA,
    ],
    compiler_params=pltpu.CompilerParams(
        use_tc_tiling_on_sc=False, needs_layout_passes=False
    ),
)
def _emb_kernel(pos_hbm, table_hbm, out_hbm, pos_v, tq_v, out0_v, out1_v, sem0, sem1, sem2, sem3):
    wid = lax.axis_index("s") * 2 + lax.axis_index("c")
    j = wid % _NCOORD
    sl = wid // _NCOORD
    base = sl * _PW

    tdma = pltpu.async_copy(table_hbm.at[:, pl.ds(j * _F4, _F4)], tq_v, sem0)
    pdma = pltpu.async_copy(pos_hbm.at[j, pl.ds(base, _PW)], pos_v, sem1)
    tdma.wait()
    pdma.wait()

    lanes = lax.iota(jnp.int32, _L)
    half = _PW // 2

    handles = []
    for c, (buf, sem) in enumerate(((out0_v, sem2), (out1_v, sem3))):

        @plsc.parallel_loop(0, _G // 2, step=1, unroll=4)
        def body(g, _c=c, _buf=buf):
            row = g * _L + lanes
            pv = pos_v[pl.ds(_c * half + g * _L, _L)]
            d = pv * float(_EMB)
            dc = jnp.minimum(jnp.maximum(d, 0.0), float(_EMB - 1))
            li = dc.astype(jnp.int32)
            lw = d - li.astype(jnp.float32)
            rw = 1.0 - lw
            ri = jnp.minimum(li + 1, _EMB - 1)
            for k in range(0, _F4, 4):
                kvs = [lanes ^ (k + i) for i in range(4)]
                lvs = [plsc.load_gather(tq_v, [li, kv]) for kv in kvs]
                rvs = [plsc.load_gather(tq_v, [ri, kv]) for kv in kvs]
                os_ = [lw * a + rw * b for a, b in zip(lvs, rvs)]
                for kv, o in zip(kvs, os_):
                    plsc.store_scatter(_buf, [row, kv], o)

        handles.append(
            pltpu.async_copy(
                buf,
                out_hbm.at[pl.ds(base + c * half, half), pl.ds(j * _F4, _F4)],
                sem,
            )
        )
    for h in handles:
        h.wait()


def kernel(seq_positions, lookup_weight):
    pos_t = seq_positions.T
    tab128 = lookup_weight.reshape(_EMB, _NCOORD * _F4)
    return _emb_kernel(pos_t, tab128)
